# baseline (speedup 1.0000x reference)
import functools

import jax
import jax.numpy as jnp
from jax import lax
from jax.experimental import pallas as pl
from jax.experimental.pallas import tpu as pltpu
from jax.experimental.pallas import tpu_sc as plsc

N = 10000
E = 320000
NF = 128
DIM = 128
B = 64
STEPS = 3

NPAD = 10240
RB = 1024
NW = 32
EW = E // NW
C = 80
NCHUNK = EW // C
ROWS_PER_TILE = NPAD // 16
HF = 64


def _softplus16(s):
    t = jnp.exp(-jnp.abs(s))
    zz = t / (2.0 + t)
    z2 = zz * zz
    p = zz * (2.0 + z2 * (2.0 / 3.0 + z2 * (2.0 / 5.0 + z2 * (2.0 / 7.0))))
    return jnp.maximum(s, 0.0) + p


NH = NPAD // 2
RPT2 = NH // 16


def _sc_conv_body(tdA_hbm, tsA_hbm, tdB_hbm, tsB_hbm, dst_hbm, src_hbm,
                  out_hbm, dsti0, srci0, dsti1, srci1, dsti2, parb,
                  bufD0, bufS0, bufD1, bufS1, mbuf, agg_sh,
                  semD0, semS0, semD1, semS1):
    cid = lax.axis_index("c")
    sid = lax.axis_index("s")
    wid = sid * 2 + cid
    ebase = wid * EW
    bufs = ((dsti0, srci0, bufD0, bufS0, semD0, semS0),
            (dsti1, srci1, bufD1, bufS1, semD1, semS1))

    def zrow(e, carry):
        z16 = jnp.zeros((16,), jnp.float32)
        for j in range(8):
            mbuf[e, pl.ds(j * 16, 16)] = z16
        return carry
    lax.fori_loop(0, C, zrow, 0)

    def issue(ci, b, td_hbm, ts_hbm):
        di, si, bD, bS, sD, sS = bufs[b]
        base = pl.multiple_of(ebase + ci * C, 8)
        pltpu.sync_copy(dst_hbm.at[pl.ds(base, C)], di)
        pltpu.sync_copy(src_hbm.at[pl.ds(base, C)], si)
        pltpu.async_copy(td_hbm.at[di], bD, sD)
        pltpu.async_copy(ts_hbm.at[si], bS, sS)

    def drain(b, td_hbm, ts_hbm):
        di, si, bD, bS, sD, sS = bufs[b]
        pltpu.make_async_copy(td_hbm.at[di], bD, sD).wait()
        pltpu.make_async_copy(ts_hbm.at[si], bS, sS).wait()

    def work(b):
        di, si, bD, bS, sD, sS = bufs[b]
        for k2 in range(C // 16):
            dv = di[pl.ds(k2 * 16, 16)]
            dsti2[pl.ds(k2 * 16, 16)] = lax.shift_right_logical(dv, 1)
            parb[pl.ds(k2 * 16, 16)] = (dv & 1).astype(jnp.float32)

        @plsc.parallel_loop(0, C, unroll=2)
        def edge(e):
            pv = parb[pl.ds(e, 16)]
            par = lax.broadcast(pv[0], (16,))
            npar = 1.0 - par
            for j in range(HF // 16):
                f = bD[e, pl.ds(j * 16, 16)] + bS[e, pl.ds(j * 16, 16)]
                s = (bD[e, pl.ds(HF + j * 16, 16)]
                     + bS[e, pl.ds(HF + j * 16, 16)])
                m = _softplus16(s) / (1.0 + jnp.exp(-f))
                mbuf[e, pl.ds(j * 16, 16)] = m * npar
                mbuf[e, pl.ds(HF + j * 16, 16)] = m * par
        pltpu.sync_copy(mbuf, agg_sh.at[dsti2], add=True)

    for p, (td_hbm, ts_hbm) in enumerate(((tdA_hbm, tsA_hbm),
                                          (tdB_hbm, tsB_hbm))):
        for k in range(RPT2 // C):
            pltpu.sync_copy(mbuf, agg_sh.at[pl.ds(sid * RPT2 + k * C, C)])
        pltpu.sync_copy(mbuf, agg_sh.at[pl.ds(sid * RPT2, C)])
        plsc.subcore_barrier()

        issue(0, 0, td_hbm, ts_hbm)

        def chunk2(i, carry):
            issue(2 * i + 1, 1, td_hbm, ts_hbm)
            drain(0, td_hbm, ts_hbm)
            work(0)
            nxt = jnp.minimum(2 * i + 2, NCHUNK - 1)
            issue(nxt, 0, td_hbm, ts_hbm)
            drain(1, td_hbm, ts_hbm)
            work(1)
            return carry

        lax.fori_loop(0, NCHUNK // 2, chunk2, 0)
        drain(0, td_hbm, ts_hbm)
        work(0)
        plsc.subcore_barrier()

        for k in range(RPT2 // C):
            r0 = sid * RPT2 + k * C
            pltpu.sync_copy(agg_sh.at[pl.ds(r0, C)],
                            out_hbm.at[cid, p, pl.ds(r0, C)])
        if p == 0:
            plsc.subcore_barrier()
            lax.fori_loop(0, C, zrow, 0)


@functools.lru_cache(maxsize=1)
def _sc_conv_fn():
    return pl.kernel(
        _sc_conv_body,
        out_type=jax.ShapeDtypeStruct((2, 2, NH, 128), jnp.float32),
        mesh=plsc.VectorSubcoreMesh(core_axis_name="c", subcore_axis_name="s",
                                    num_cores=2, num_subcores=16),
        scratch_types=[
            pltpu.VMEM((C,), jnp.int32),
            pltpu.VMEM((C,), jnp.int32),
            pltpu.VMEM((C,), jnp.int32),
            pltpu.VMEM((C,), jnp.int32),
            pltpu.VMEM((C,), jnp.int32),
            pltpu.VMEM((C + 16,), jnp.float32),
            pltpu.VMEM((C, 128), jnp.float32),
            pltpu.VMEM((C, 128), jnp.float32),
            pltpu.VMEM((C, 128), jnp.float32),
            pltpu.VMEM((C, 128), jnp.float32),
            pltpu.VMEM((C, 128), jnp.float32),
            pltpu.VMEM_SHARED((NH, 128), jnp.float32),
            pltpu.SemaphoreType.DMA,
            pltpu.SemaphoreType.DMA,
            pltpu.SemaphoreType.DMA,
            pltpu.SemaphoreType.DMA,
        ],
    )


def _sc_conv(tdA, tsA, tdB, tsB, dst, src):
    out = _sc_conv_fn()(tdA, tsA, tdB, tsB, dst, src)
    return out.reshape(2, 2, NPAD, HF)


def _dot(a, b):
    return jnp.dot(a, b, preferred_element_type=jnp.float32)


def _agg_sum(agg):
    return jnp.concatenate([agg[0, 0] + agg[1, 0], agg[0, 1] + agg[1, 1]],
                           axis=1)


_TBL = lambda i: (0, 0)


def _table_specs():
    return [
        pl.BlockSpec((DIM, 128), _TBL),
        pl.BlockSpec((1, 128), _TBL),
        pl.BlockSpec((DIM, 128), _TBL),
        pl.BlockSpec((DIM, 128), _TBL),
        pl.BlockSpec((1, 128), _TBL),
        pl.BlockSpec((DIM, 128), _TBL),
    ]


def _table_outs():
    specs = [pl.BlockSpec((RB, 128), lambda i: (i, 0)) for _ in range(4)]
    shapes = [jax.ShapeDtypeStruct((NPAD, 128), jnp.float32) for _ in range(4)]
    return specs, shapes


def _emit_tables(o, wdA, bdA, wsA, wdB, bdB, wsB, tdA, tsA, tdB, tsB):
    tdA[...] = _dot(o, wdA[...]) + bdA[...]
    tsA[...] = _dot(o, wsA[...])
    tdB[...] = _dot(o, wdB[...]) + bdB[...]
    tsB[...] = _dot(o, wsB[...])


def _tc_tables1_body(x_ref, wdA, bdA, wsA, wdB, bdB, wsB,
                     tdA, tsA, tdB, tsB):
    _emit_tables(x_ref[...], wdA, bdA, wsA, wdB, bdB, wsB, tdA, tsA, tdB, tsB)


def _tc_tables1(x, tw):
    ospecs, oshapes = _table_outs()
    return pl.pallas_call(
        _tc_tables1_body,
        grid=(NPAD // RB,),
        in_specs=[pl.BlockSpec((RB, NF), lambda i: (i, 0))] + _table_specs(),
        out_specs=ospecs,
        out_shape=oshapes,
    )(x, *tw)


def _tc_post1_body(x_ref, agg_ref, w0_ref, b0_ref, w1_ref, b1_ref,
                   wdA, bdA, wsA, wdB, bdB, wsB,
                   out_ref, tdA, tsA, tdB, tsB):
    o = jnp.maximum(x_ref[...] + _agg_sum(agg_ref[...]), 0.0)
    o = jnp.maximum(_dot(o, w0_ref[...]) + b0_ref[...], 0.0)
    o = jnp.maximum(_dot(o, w1_ref[...]) + b1_ref[...], 0.0)
    out_ref[...] = o
    _emit_tables(o, wdA, bdA, wsA, wdB, bdB, wsB, tdA, tsA, tdB, tsB)


def _tc_post1(x, agg, W0, b0, W1, b1, tw):
    ospecs, oshapes = _table_outs()
    return pl.pallas_call(
        _tc_post1_body,
        grid=(NPAD // RB,),
        in_specs=[
            pl.BlockSpec((RB, NF), lambda i: (i, 0)),
            pl.BlockSpec((2, 2, RB, HF), lambda i: (0, 0, i, 0)),
            pl.BlockSpec((NF, DIM), _TBL),
            pl.BlockSpec((1, DIM), _TBL),
            pl.BlockSpec((DIM, DIM), _TBL),
            pl.BlockSpec((1, DIM), _TBL),
        ] + _table_specs(),
        out_specs=[pl.BlockSpec((RB, DIM), lambda i: (i, 0))] + ospecs,
        out_shape=[jax.ShapeDtypeStruct((NPAD, DIM), jnp.float32)] + oshapes,
    )(x, agg, W0, b0, W1, b1, *tw)


def _gru(m, h, wih_ref, whh_ref, bih_ref, bhh_ref):
    gi = _dot(m, wih_ref[...]) + bih_ref[...]
    gh = _dot(h, whh_ref[...]) + bhh_ref[...]
    r = jax.nn.sigmoid(gi[:, :DIM] + gh[:, :DIM])
    z = jax.nn.sigmoid(gi[:, DIM:2 * DIM] + gh[:, DIM:2 * DIM])
    n = jnp.tanh(gi[:, 2 * DIM:] + r * gh[:, 2 * DIM:])
    return (1.0 - z) * n + z * h


def _tc_gru_tables_body(cin_ref, h_ref, agg_ref, wih_ref, whh_ref, bih_ref,
                        bhh_ref, wdA, bdA, wsA, wdB, bdB, wsB,
                        hout_ref, tdA, tsA, tdB, tsB):
    m = cin_ref[...] + _agg_sum(agg_ref[...])
    h2 = _gru(m, h_ref[...], wih_ref, whh_ref, bih_ref, bhh_ref)
    hout_ref[...] = h2
    _emit_tables(h2, wdA, bdA, wsA, wdB, bdB, wsB, tdA, tsA, tdB, tsB)


def _tc_gru_tables(cin, h, agg, Wih, Whh, bih, bhh, tw):
    ospecs, oshapes = _table_outs()
    return pl.pallas_call(
        _tc_gru_tables_body,
        grid=(NPAD // RB,),
        in_specs=[
            pl.BlockSpec((RB, DIM), lambda i: (i, 0)),
            pl.BlockSpec((RB, DIM), lambda i: (i, 0)),
            pl.BlockSpec((2, 2, RB, HF), lambda i: (0, 0, i, 0)),
            pl.BlockSpec((DIM, 3 * DIM), _TBL),
            pl.BlockSpec((DIM, 3 * DIM), _TBL),
            pl.BlockSpec((1, 3 * DIM), _TBL),
            pl.BlockSpec((1, 3 * DIM), _TBL),
        ] + _table_specs(),
        out_specs=[pl.BlockSpec((RB, DIM), lambda i: (i, 0))] + ospecs,
        out_shape=[jax.ShapeDtypeStruct((NPAD, DIM), jnp.float32)] + oshapes,
    )(cin, h, agg, Wih, Whh, bih, bhh, *tw)


def _tc_gru_connect_body(cin_ref, h_ref, agg_ref, wih_ref, whh_ref, bih_ref,
                         bhh_ref, wc_ref, bc_ref, outc_ref):
    m = cin_ref[...] + _agg_sum(agg_ref[...])
    h2 = _gru(m, h_ref[...], wih_ref, whh_ref, bih_ref, bhh_ref)
    outc_ref[...] = jnp.maximum(_dot(h2, wc_ref[...]) + bc_ref[...], 0.0)


def _tc_gru_connect(cin, h, agg, Wih, Whh, bih, bhh, Wceff, bc):
    return pl.pallas_call(
        _tc_gru_connect_body,
        grid=(NPAD // RB,),
        in_specs=[
            pl.BlockSpec((RB, DIM), lambda i: (i, 0)),
            pl.BlockSpec((RB, DIM), lambda i: (i, 0)),
            pl.BlockSpec((2, 2, RB, HF), lambda i: (0, 0, i, 0)),
            pl.BlockSpec((DIM, 3 * DIM), _TBL),
            pl.BlockSpec((DIM, 3 * DIM), _TBL),
            pl.BlockSpec((1, 3 * DIM), _TBL),
            pl.BlockSpec((1, 3 * DIM), _TBL),
            pl.BlockSpec((DIM, DIM), _TBL),
            pl.BlockSpec((1, DIM), _TBL),
        ],
        out_specs=[pl.BlockSpec((RB, DIM), lambda i: (i, 0))],
        out_shape=[jax.ShapeDtypeStruct((NPAD, DIM), jnp.float32)],
    )(cin, h, agg, Wih, Whh, bih, bhh, Wceff, bc)[0]


def _tc_set2set_body(xc_ref, b_ref, wih_ref, whh_ref, bih_ref, bhh_ref,
                     w2_ref, b2_ref, w3_ref, b3_ref, out_ref):
    xc = xc_ref[...]
    bvec = b_ref[...]
    ids = lax.broadcasted_iota(jnp.int32, (NPAD, B), 1)
    oneh = (bvec == ids).astype(jnp.float32)
    valid = (bvec < B).astype(jnp.float32)

    h = jnp.zeros((B, DIM), jnp.float32)
    c = jnp.zeros((B, DIM), jnp.float32)
    q_star = jnp.zeros((B, 2 * DIM), jnp.float32)
    dn = (((0,), (0,)), ((), ()))

    for _ in range(STEPS):
        g = (_dot(q_star, wih_ref[...]) + bih_ref[...]
             + _dot(h, whh_ref[...]) + bhh_ref[...])
        i = jax.nn.sigmoid(g[:, :DIM])
        f = jax.nn.sigmoid(g[:, DIM:2 * DIM])
        gg = jnp.tanh(g[:, 2 * DIM:3 * DIM])
        o = jax.nn.sigmoid(g[:, 3 * DIM:])
        c = f * c + i * gg
        h = o * jnp.tanh(c)
        q = h
        qb = _dot(oneh, q)
        e = jnp.sum(xc * qb, axis=1, keepdims=True)
        e_where = jnp.where(oneh > 0.0, e, -1e30)
        emax = jnp.max(e_where, axis=0, keepdims=True)
        emax = jnp.where(emax < -1e29, 0.0, emax)
        emax_n = _dot(oneh, emax.T)
        a = jnp.exp(e - emax_n) * valid
        den = lax.dot_general(oneh, a, dn,
                              preferred_element_type=jnp.float32)
        den_n = _dot(oneh, den)
        aw = a / (den_n + 1e-16)
        r = lax.dot_general(oneh, aw * xc, dn,
                            preferred_element_type=jnp.float32)
        q_star = jnp.concatenate([q, r], axis=1)

    o2 = jnp.maximum(_dot(q_star, w2_ref[...]) + b2_ref[...], 0.0)
    out_ref[...] = _dot(o2, w3_ref[...]) + b3_ref[...]


def _tc_set2set(xc, batch_pad, Wih, Whh, bih, bhh, W2, b2, W3, b3):
    return pl.pallas_call(
        _tc_set2set_body,
        in_specs=[
            pl.BlockSpec((NPAD, DIM), lambda: (0, 0)),
            pl.BlockSpec((NPAD, 1), lambda: (0, 0)),
            pl.BlockSpec((2 * DIM, 4 * DIM), lambda: (0, 0)),
            pl.BlockSpec((DIM, 4 * DIM), lambda: (0, 0)),
            pl.BlockSpec((1, 4 * DIM), lambda: (0, 0)),
            pl.BlockSpec((1, 4 * DIM), lambda: (0, 0)),
            pl.BlockSpec((2 * DIM, NF), lambda: (0, 0)),
            pl.BlockSpec((1, NF), lambda: (0, 0)),
            pl.BlockSpec((NF, 1), lambda: (0, 0)),
            pl.BlockSpec((1, 1), lambda: (0, 0)),
        ],
        out_specs=[pl.BlockSpec((B, 1), lambda: (0, 0))],
        out_shape=[jax.ShapeDtypeStruct((B, 1), jnp.float32)],
    )(xc, batch_pad, Wih, Whh, bih, bhh, W2, b2, W3, b3)[0]


def _split_table_weights(Wf, bf, Ws, bs, d):
    def phase(lo, hi):
        Wd = jnp.concatenate([Wf[:d, lo:hi], Ws[:d, lo:hi]], axis=1)
        Wsr = jnp.concatenate([Wf[d:, lo:hi], Ws[d:, lo:hi]], axis=1)
        bd = jnp.concatenate([bf[lo:hi], bs[lo:hi]]).reshape(1, 128)
        return Wd, bd, Wsr
    WdA, bdA, WsrA = phase(0, HF)
    WdB, bdB, WsrB = phase(HF, 2 * HF)
    return (WdA, bdA, WsrA, WdB, bdB, WsrB)


def kernel(x, Wf1, bf1, Ws1, bs1, W0, b0, W1, b1, Wf2, bf2, Ws2, bs2,
           Wf3, bf3, Ws3, bs3, g1_Wih, g1_Whh, g1_bih, g1_bhh,
           g2_Wih, g2_Whh, g2_bih, g2_bhh, ls_Wih, ls_Whh, ls_bih, ls_bhh,
           Wc, bc, W2, b2, W3, b3, edge_index, batch):
    f32 = jnp.float32
    x_pad = jnp.concatenate([x, jnp.zeros((NPAD - N, NF), f32)], axis=0)
    src = edge_index[0].astype(jnp.int32)
    dst = edge_index[1].astype(jnp.int32)
    batch_pad = jnp.concatenate(
        [batch.astype(jnp.int32), jnp.full((NPAD - N,), B, jnp.int32)]
    ).reshape(NPAD, 1)

    tw1 = _split_table_weights(Wf1, bf1, Ws1, bs1, NF)
    tw2 = _split_table_weights(Wf2, bf2, Ws2, bs2, DIM)
    Wceff = Wc[:DIM] + Wc[DIM:]
    r2 = lambda v: v.reshape(1, -1)

    tabs = _tc_tables1(x_pad, tw1)
    agg = _sc_conv(tabs[0], tabs[1], tabs[2], tabs[3], dst, src)
    out, *tabs = _tc_post1(x_pad, agg, W0, r2(b0), W1, r2(b1), tw2)
    h = out
    for it in range(3):
        agg = _sc_conv(tabs[0], tabs[1], tabs[2], tabs[3], dst, src)
        if it < 2:
            h, *tabs = _tc_gru_tables(out, h, agg, g1_Wih, g1_Whh,
                                      r2(g1_bih), r2(g1_bhh), tw2)
            out = h
        else:
            outc = _tc_gru_connect(out, h, agg, g1_Wih, g1_Whh,
                                   r2(g1_bih), r2(g1_bhh), Wceff, r2(bc))
    res = _tc_set2set(outc, batch_pad, ls_Wih, ls_Whh, r2(ls_bih), r2(ls_bhh),
                      W2, r2(b2), W3, r2(b3))
    return res.reshape(-1)


# speedup vs baseline: 1.0067x; 1.0067x over previous
import functools

import jax
import jax.numpy as jnp
from jax import lax
from jax.experimental import pallas as pl
from jax.experimental.pallas import tpu as pltpu
from jax.experimental.pallas import tpu_sc as plsc

N = 10000
E = 320000
NF = 128
DIM = 128
B = 64
STEPS = 3

NPAD = 10240
RB = 1024
NW = 32
EW = E // NW
C = 80
NCHUNK = EW // C
ROWS_PER_TILE = NPAD // 16
HF = 64


def _softplus16(s):
    t = jnp.exp(-jnp.abs(s))
    zz = t / (2.0 + t)
    z2 = zz * zz
    p = zz * (2.0 + z2 * (2.0 / 3.0 + z2 * (2.0 / 5.0 + z2 * (2.0 / 7.0 + z2 * (2.0 / 9.0)))))
    return jnp.maximum(s, 0.0) + p


NH = NPAD // 2
RPT2 = NH // 16


def _sc_conv_body(tdA_hbm, tsA_hbm, tdB_hbm, tsB_hbm, dst_hbm, src_hbm,
                  out_hbm, dsti0, srci0, dsti1, srci1, dsti2, parb,
                  bufD0, bufS0, bufD1, bufS1, mbuf, agg_sh,
                  semD0, semS0, semD1, semS1):
    cid = lax.axis_index("c")
    sid = lax.axis_index("s")
    wid = sid * 2 + cid
    ebase = wid * EW
    bufs = ((dsti0, srci0, bufD0, bufS0, semD0, semS0),
            (dsti1, srci1, bufD1, bufS1, semD1, semS1))

    def zrow(e, carry):
        z16 = jnp.zeros((16,), jnp.float32)
        for j in range(8):
            mbuf[e, pl.ds(j * 16, 16)] = z16
        return carry
    lax.fori_loop(0, C, zrow, 0)

    def issue(ci, b, td_hbm, ts_hbm):
        di, si, bD, bS, sD, sS = bufs[b]
        base = pl.multiple_of(ebase + ci * C, 8)
        pltpu.sync_copy(dst_hbm.at[pl.ds(base, C)], di)
        pltpu.sync_copy(src_hbm.at[pl.ds(base, C)], si)
        pltpu.async_copy(td_hbm.at[di], bD, sD)
        pltpu.async_copy(ts_hbm.at[si], bS, sS)

    def drain(b, td_hbm, ts_hbm):
        di, si, bD, bS, sD, sS = bufs[b]
        pltpu.make_async_copy(td_hbm.at[di], bD, sD).wait()
        pltpu.make_async_copy(ts_hbm.at[si], bS, sS).wait()

    def work(b):
        di, si, bD, bS, sD, sS = bufs[b]
        for k2 in range(C // 16):
            dv = di[pl.ds(k2 * 16, 16)]
            dsti2[pl.ds(k2 * 16, 16)] = lax.shift_right_logical(dv, 1)
            parb[pl.ds(k2 * 16, 16)] = (dv & 1).astype(jnp.float32)

        @plsc.parallel_loop(0, C)
        def edge(e):
            pv = parb[pl.ds(e, 16)]
            par = lax.broadcast(pv[0], (16,))
            npar = 1.0 - par
            for j in range(HF // 16):
                f = bD[e, pl.ds(j * 16, 16)] + bS[e, pl.ds(j * 16, 16)]
                s = (bD[e, pl.ds(HF + j * 16, 16)]
                     + bS[e, pl.ds(HF + j * 16, 16)])
                m = _softplus16(s) / (1.0 + jnp.exp(-f))
                mbuf[e, pl.ds(j * 16, 16)] = m * npar
                mbuf[e, pl.ds(HF + j * 16, 16)] = m * par
        pltpu.sync_copy(mbuf, agg_sh.at[dsti2], add=True)

    for p, (td_hbm, ts_hbm) in enumerate(((tdA_hbm, tsA_hbm),
                                          (tdB_hbm, tsB_hbm))):
        for k in range(RPT2 // C):
            pltpu.sync_copy(mbuf, agg_sh.at[pl.ds(sid * RPT2 + k * C, C)])
        pltpu.sync_copy(mbuf, agg_sh.at[pl.ds(sid * RPT2, C)])
        plsc.subcore_barrier()

        issue(0, 0, td_hbm, ts_hbm)

        def chunk2(i, carry):
            issue(2 * i + 1, 1, td_hbm, ts_hbm)
            drain(0, td_hbm, ts_hbm)
            work(0)
            nxt = jnp.minimum(2 * i + 2, NCHUNK - 1)
            issue(nxt, 0, td_hbm, ts_hbm)
            drain(1, td_hbm, ts_hbm)
            work(1)
            return carry

        lax.fori_loop(0, NCHUNK // 2, chunk2, 0)
        drain(0, td_hbm, ts_hbm)
        work(0)
        plsc.subcore_barrier()

        for k in range(RPT2 // C):
            r0 = sid * RPT2 + k * C
            pltpu.sync_copy(agg_sh.at[pl.ds(r0, C)],
                            out_hbm.at[cid, p, pl.ds(r0, C)])
        if p == 0:
            plsc.subcore_barrier()
            lax.fori_loop(0, C, zrow, 0)


@functools.lru_cache(maxsize=1)
def _sc_conv_fn():
    return pl.kernel(
        _sc_conv_body,
        out_type=jax.ShapeDtypeStruct((2, 2, NH, 128), jnp.float32),
        mesh=plsc.VectorSubcoreMesh(core_axis_name="c", subcore_axis_name="s",
                                    num_cores=2, num_subcores=16),
        scratch_types=[
            pltpu.VMEM((C,), jnp.int32),
            pltpu.VMEM((C,), jnp.int32),
            pltpu.VMEM((C,), jnp.int32),
            pltpu.VMEM((C,), jnp.int32),
            pltpu.VMEM((C,), jnp.int32),
            pltpu.VMEM((C + 16,), jnp.float32),
            pltpu.VMEM((C, 128), jnp.float32),
            pltpu.VMEM((C, 128), jnp.float32),
            pltpu.VMEM((C, 128), jnp.float32),
            pltpu.VMEM((C, 128), jnp.float32),
            pltpu.VMEM((C, 128), jnp.float32),
            pltpu.VMEM_SHARED((NH, 128), jnp.float32),
            pltpu.SemaphoreType.DMA,
            pltpu.SemaphoreType.DMA,
            pltpu.SemaphoreType.DMA,
            pltpu.SemaphoreType.DMA,
        ],
    )


def _sc_conv(tdA, tsA, tdB, tsB, dst, src):
    out = _sc_conv_fn()(tdA, tsA, tdB, tsB, dst, src)
    return out.reshape(2, 2, NPAD, HF)


def _dot(a, b):
    return jnp.dot(a, b, preferred_element_type=jnp.float32)


def _agg_sum(agg):
    return jnp.concatenate([agg[0, 0] + agg[1, 0], agg[0, 1] + agg[1, 1]],
                           axis=1)


_TBL = lambda i: (0, 0)


def _table_specs():
    return [
        pl.BlockSpec((DIM, 128), _TBL),
        pl.BlockSpec((1, 128), _TBL),
        pl.BlockSpec((DIM, 128), _TBL),
        pl.BlockSpec((DIM, 128), _TBL),
        pl.BlockSpec((1, 128), _TBL),
        pl.BlockSpec((DIM, 128), _TBL),
    ]


def _table_outs():
    specs = [pl.BlockSpec((RB, 128), lambda i: (i, 0)) for _ in range(4)]
    shapes = [jax.ShapeDtypeStruct((NPAD, 128), jnp.float32) for _ in range(4)]
    return specs, shapes


def _emit_tables(o, wdA, bdA, wsA, wdB, bdB, wsB, tdA, tsA, tdB, tsB):
    tdA[...] = _dot(o, wdA[...]) + bdA[...]
    tsA[...] = _dot(o, wsA[...])
    tdB[...] = _dot(o, wdB[...]) + bdB[...]
    tsB[...] = _dot(o, wsB[...])


def _tc_tables1_body(x_ref, wdA, bdA, wsA, wdB, bdB, wsB,
                     tdA, tsA, tdB, tsB):
    _emit_tables(x_ref[...], wdA, bdA, wsA, wdB, bdB, wsB, tdA, tsA, tdB, tsB)


def _tc_tables1(x, tw):
    ospecs, oshapes = _table_outs()
    return pl.pallas_call(
        _tc_tables1_body,
        grid=(NPAD // RB,),
        in_specs=[pl.BlockSpec((RB, NF), lambda i: (i, 0))] + _table_specs(),
        out_specs=ospecs,
        out_shape=oshapes,
    )(x, *tw)


def _tc_post1_body(x_ref, agg_ref, w0_ref, b0_ref, w1_ref, b1_ref,
                   wdA, bdA, wsA, wdB, bdB, wsB,
                   out_ref, tdA, tsA, tdB, tsB):
    o = jnp.maximum(x_ref[...] + _agg_sum(agg_ref[...]), 0.0)
    o = jnp.maximum(_dot(o, w0_ref[...]) + b0_ref[...], 0.0)
    o = jnp.maximum(_dot(o, w1_ref[...]) + b1_ref[...], 0.0)
    out_ref[...] = o
    _emit_tables(o, wdA, bdA, wsA, wdB, bdB, wsB, tdA, tsA, tdB, tsB)


def _tc_post1(x, agg, W0, b0, W1, b1, tw):
    ospecs, oshapes = _table_outs()
    return pl.pallas_call(
        _tc_post1_body,
        grid=(NPAD // RB,),
        in_specs=[
            pl.BlockSpec((RB, NF), lambda i: (i, 0)),
            pl.BlockSpec((2, 2, RB, HF), lambda i: (0, 0, i, 0)),
            pl.BlockSpec((NF, DIM), _TBL),
            pl.BlockSpec((1, DIM), _TBL),
            pl.BlockSpec((DIM, DIM), _TBL),
            pl.BlockSpec((1, DIM), _TBL),
        ] + _table_specs(),
        out_specs=[pl.BlockSpec((RB, DIM), lambda i: (i, 0))] + ospecs,
        out_shape=[jax.ShapeDtypeStruct((NPAD, DIM), jnp.float32)] + oshapes,
    )(x, agg, W0, b0, W1, b1, *tw)


def _gru(m, h, wih_ref, whh_ref, bih_ref, bhh_ref):
    gi = _dot(m, wih_ref[...]) + bih_ref[...]
    gh = _dot(h, whh_ref[...]) + bhh_ref[...]
    r = jax.nn.sigmoid(gi[:, :DIM] + gh[:, :DIM])
    z = jax.nn.sigmoid(gi[:, DIM:2 * DIM] + gh[:, DIM:2 * DIM])
    n = jnp.tanh(gi[:, 2 * DIM:] + r * gh[:, 2 * DIM:])
    return (1.0 - z) * n + z * h


def _tc_gru_tables_body(cin_ref, h_ref, agg_ref, wih_ref, whh_ref, bih_ref,
                        bhh_ref, wdA, bdA, wsA, wdB, bdB, wsB,
                        hout_ref, tdA, tsA, tdB, tsB):
    m = cin_ref[...] + _agg_sum(agg_ref[...])
    h2 = _gru(m, h_ref[...], wih_ref, whh_ref, bih_ref, bhh_ref)
    hout_ref[...] = h2
    _emit_tables(h2, wdA, bdA, wsA, wdB, bdB, wsB, tdA, tsA, tdB, tsB)


def _tc_gru_tables(cin, h, agg, Wih, Whh, bih, bhh, tw):
    ospecs, oshapes = _table_outs()
    return pl.pallas_call(
        _tc_gru_tables_body,
        grid=(NPAD // RB,),
        in_specs=[
            pl.BlockSpec((RB, DIM), lambda i: (i, 0)),
            pl.BlockSpec((RB, DIM), lambda i: (i, 0)),
            pl.BlockSpec((2, 2, RB, HF), lambda i: (0, 0, i, 0)),
            pl.BlockSpec((DIM, 3 * DIM), _TBL),
            pl.BlockSpec((DIM, 3 * DIM), _TBL),
            pl.BlockSpec((1, 3 * DIM), _TBL),
            pl.BlockSpec((1, 3 * DIM), _TBL),
        ] + _table_specs(),
        out_specs=[pl.BlockSpec((RB, DIM), lambda i: (i, 0))] + ospecs,
        out_shape=[jax.ShapeDtypeStruct((NPAD, DIM), jnp.float32)] + oshapes,
    )(cin, h, agg, Wih, Whh, bih, bhh, *tw)


def _tc_gru_connect_body(cin_ref, h_ref, agg_ref, wih_ref, whh_ref, bih_ref,
                         bhh_ref, wc_ref, bc_ref, outc_ref):
    m = cin_ref[...] + _agg_sum(agg_ref[...])
    h2 = _gru(m, h_ref[...], wih_ref, whh_ref, bih_ref, bhh_ref)
    outc_ref[...] = jnp.maximum(_dot(h2, wc_ref[...]) + bc_ref[...], 0.0)


def _tc_gru_connect(cin, h, agg, Wih, Whh, bih, bhh, Wceff, bc):
    return pl.pallas_call(
        _tc_gru_connect_body,
        grid=(NPAD // RB,),
        in_specs=[
            pl.BlockSpec((RB, DIM), lambda i: (i, 0)),
            pl.BlockSpec((RB, DIM), lambda i: (i, 0)),
            pl.BlockSpec((2, 2, RB, HF), lambda i: (0, 0, i, 0)),
            pl.BlockSpec((DIM, 3 * DIM), _TBL),
            pl.BlockSpec((DIM, 3 * DIM), _TBL),
            pl.BlockSpec((1, 3 * DIM), _TBL),
            pl.BlockSpec((1, 3 * DIM), _TBL),
            pl.BlockSpec((DIM, DIM), _TBL),
            pl.BlockSpec((1, DIM), _TBL),
        ],
        out_specs=[pl.BlockSpec((RB, DIM), lambda i: (i, 0))],
        out_shape=[jax.ShapeDtypeStruct((NPAD, DIM), jnp.float32)],
    )(cin, h, agg, Wih, Whh, bih, bhh, Wceff, bc)[0]


def _tc_set2set_body(xc_ref, b_ref, wih_ref, whh_ref, bih_ref, bhh_ref,
                     w2_ref, b2_ref, w3_ref, b3_ref, out_ref):
    xc = xc_ref[...]
    bvec = b_ref[...]
    ids = lax.broadcasted_iota(jnp.int32, (NPAD, B), 1)
    oneh = (bvec == ids).astype(jnp.float32)
    valid = (bvec < B).astype(jnp.float32)

    h = jnp.zeros((B, DIM), jnp.float32)
    c = jnp.zeros((B, DIM), jnp.float32)
    q_star = jnp.zeros((B, 2 * DIM), jnp.float32)
    dn = (((0,), (0,)), ((), ()))

    for _ in range(STEPS):
        g = (_dot(q_star, wih_ref[...]) + bih_ref[...]
             + _dot(h, whh_ref[...]) + bhh_ref[...])
        i = jax.nn.sigmoid(g[:, :DIM])
        f = jax.nn.sigmoid(g[:, DIM:2 * DIM])
        gg = jnp.tanh(g[:, 2 * DIM:3 * DIM])
        o = jax.nn.sigmoid(g[:, 3 * DIM:])
        c = f * c + i * gg
        h = o * jnp.tanh(c)
        q = h
        qb = _dot(oneh, q)
        e = jnp.sum(xc * qb, axis=1, keepdims=True)
        e_where = jnp.where(oneh > 0.0, e, -1e30)
        emax = jnp.max(e_where, axis=0, keepdims=True)
        emax = jnp.where(emax < -1e29, 0.0, emax)
        emax_n = _dot(oneh, emax.T)
        a = jnp.exp(e - emax_n) * valid
        den = lax.dot_general(oneh, a, dn,
                              preferred_element_type=jnp.float32)
        den_n = _dot(oneh, den)
        aw = a / (den_n + 1e-16)
        r = lax.dot_general(oneh, aw * xc, dn,
                            preferred_element_type=jnp.float32)
        q_star = jnp.concatenate([q, r], axis=1)

    o2 = jnp.maximum(_dot(q_star, w2_ref[...]) + b2_ref[...], 0.0)
    out_ref[...] = _dot(o2, w3_ref[...]) + b3_ref[...]


def _tc_set2set(xc, batch_pad, Wih, Whh, bih, bhh, W2, b2, W3, b3):
    return pl.pallas_call(
        _tc_set2set_body,
        in_specs=[
            pl.BlockSpec((NPAD, DIM), lambda: (0, 0)),
            pl.BlockSpec((NPAD, 1), lambda: (0, 0)),
            pl.BlockSpec((2 * DIM, 4 * DIM), lambda: (0, 0)),
            pl.BlockSpec((DIM, 4 * DIM), lambda: (0, 0)),
            pl.BlockSpec((1, 4 * DIM), lambda: (0, 0)),
            pl.BlockSpec((1, 4 * DIM), lambda: (0, 0)),
            pl.BlockSpec((2 * DIM, NF), lambda: (0, 0)),
            pl.BlockSpec((1, NF), lambda: (0, 0)),
            pl.BlockSpec((NF, 1), lambda: (0, 0)),
            pl.BlockSpec((1, 1), lambda: (0, 0)),
        ],
        out_specs=[pl.BlockSpec((B, 1), lambda: (0, 0))],
        out_shape=[jax.ShapeDtypeStruct((B, 1), jnp.float32)],
    )(xc, batch_pad, Wih, Whh, bih, bhh, W2, b2, W3, b3)[0]


def _split_table_weights(Wf, bf, Ws, bs, d):
    def phase(lo, hi):
        Wd = jnp.concatenate([Wf[:d, lo:hi], Ws[:d, lo:hi]], axis=1)
        Wsr = jnp.concatenate([Wf[d:, lo:hi], Ws[d:, lo:hi]], axis=1)
        bd = jnp.concatenate([bf[lo:hi], bs[lo:hi]]).reshape(1, 128)
        return Wd, bd, Wsr
    WdA, bdA, WsrA = phase(0, HF)
    WdB, bdB, WsrB = phase(HF, 2 * HF)
    return (WdA, bdA, WsrA, WdB, bdB, WsrB)


def kernel(x, Wf1, bf1, Ws1, bs1, W0, b0, W1, b1, Wf2, bf2, Ws2, bs2,
           Wf3, bf3, Ws3, bs3, g1_Wih, g1_Whh, g1_bih, g1_bhh,
           g2_Wih, g2_Whh, g2_bih, g2_bhh, ls_Wih, ls_Whh, ls_bih, ls_bhh,
           Wc, bc, W2, b2, W3, b3, edge_index, batch):
    f32 = jnp.float32
    x_pad = jnp.concatenate([x, jnp.zeros((NPAD - N, NF), f32)], axis=0)
    src = edge_index[0].astype(jnp.int32)
    dst = edge_index[1].astype(jnp.int32)
    batch_pad = jnp.concatenate(
        [batch.astype(jnp.int32), jnp.full((NPAD - N,), B, jnp.int32)]
    ).reshape(NPAD, 1)

    tw1 = _split_table_weights(Wf1, bf1, Ws1, bs1, NF)
    tw2 = _split_table_weights(Wf2, bf2, Ws2, bs2, DIM)
    Wceff = Wc[:DIM] + Wc[DIM:]
    r2 = lambda v: v.reshape(1, -1)

    tabs = _tc_tables1(x_pad, tw1)
    agg = _sc_conv(tabs[0], tabs[1], tabs[2], tabs[3], dst, src)
    out, *tabs = _tc_post1(x_pad, agg, W0, r2(b0), W1, r2(b1), tw2)
    h = out
    for it in range(3):
        agg = _sc_conv(tabs[0], tabs[1], tabs[2], tabs[3], dst, src)
        if it < 2:
            h, *tabs = _tc_gru_tables(out, h, agg, g1_Wih, g1_Whh,
                                      r2(g1_bih), r2(g1_bhh), tw2)
            out = h
        else:
            outc = _tc_gru_connect(out, h, agg, g1_Wih, g1_Whh,
                                   r2(g1_bih), r2(g1_bhh), Wceff, r2(bc))
    res = _tc_set2set(outc, batch_pad, ls_Wih, ls_Whh, r2(ls_bih), r2(ls_bhh),
                      W2, r2(b2), W3, r2(b3))
    return res.reshape(-1)
